# baseline (speedup 1.0000x reference)
import functools

import jax
import jax.numpy as jnp
from jax import lax
from jax.experimental import pallas as pl
from jax.experimental.pallas import tpu as pltpu
from jax.experimental.pallas import tpu_sc as plsc

_ROWS = 65
_D = 768
_LANES = 16
_BLKC = 128
_NBLK = _D // _BLKC


def _mean_sc(emb_table):
    info = plsc.get_sparse_core_info()
    nc, ns = info.num_cores, info.num_subcores

    mesh = plsc.VectorSubcoreMesh(core_axis_name="c", subcore_axis_name="s")

    @functools.partial(
        pl.kernel,
        mesh=mesh,
        out_type=jax.ShapeDtypeStruct((_D,), jnp.float32),
        scratch_types=[
            pltpu.VMEM((_ROWS, _BLKC), jnp.float32),
            pltpu.VMEM((_BLKC,), jnp.float32),
        ],
    )
    def mean_kernel(emb_hbm, out_hbm, colbuf, outbuf):
        wid = lax.axis_index("s") * nc + lax.axis_index("c")

        @pl.when(wid < _NBLK)
        def _():
            pltpu.sync_copy(emb_hbm.at[:, pl.ds(wid * _BLKC, _BLKC)], colbuf)
            for k in range(_BLKC // _LANES):
                sl = pl.ds(k * _LANES, _LANES)
                acc = colbuf[0, sl]
                for r in range(1, _ROWS):
                    acc = acc + colbuf[r, sl]
                outbuf[sl] = acc * (1.0 / _ROWS)
            pltpu.sync_copy(outbuf, out_hbm.at[pl.ds(wid * _BLKC, _BLKC)])

    return mean_kernel(emb_table)


def _add_body(x_ref, mean_ref, o_ref):
    o_ref[...] = x_ref[...] + mean_ref[...]


def kernel(x, emb_table):
    B, S, D = x.shape
    R = B * S
    xf = x.reshape(R, D)
    mean = _mean_sc(emb_table).reshape(1, D)
    BLK = 4096
    out = pl.pallas_call(
        _add_body,
        grid=(R // BLK,),
        in_specs=[
            pl.BlockSpec((BLK, D), lambda i: (i, 0)),
            pl.BlockSpec((1, D), lambda i: (0, 0)),
        ],
        out_specs=pl.BlockSpec((BLK, D), lambda i: (i, 0)),
        out_shape=jax.ShapeDtypeStruct((R, D), x.dtype),
    )(xf, mean)
    return out.reshape(B, S, D)


# speedup vs baseline: 2.4811x; 2.4811x over previous
import jax
import jax.numpy as jnp
from jax.experimental import pallas as pl


def _body(x_ref, emb_ref, o_ref):
    n_rows = emb_ref.shape[0]
    mean = jnp.sum(emb_ref[...], axis=0, keepdims=True) * (1.0 / n_rows)
    o_ref[...] = x_ref[...] + mean


def kernel(x, emb_table):
    B, S, D = x.shape
    R = B * S
    xf = x.reshape(R, D)
    BLK = 4096
    out = pl.pallas_call(
        _body,
        grid=(R // BLK,),
        in_specs=[
            pl.BlockSpec((BLK, D), lambda i: (i, 0)),
            pl.BlockSpec(emb_table.shape, lambda i: (0, 0)),
        ],
        out_specs=pl.BlockSpec((BLK, D), lambda i: (i, 0)),
        out_shape=jax.ShapeDtypeStruct((R, D), x.dtype),
    )(xf, emb_table)
    return out.reshape(B, S, D)
